# baseline (speedup 1.0000x reference)
import functools

import jax
import jax.numpy as jnp
from jax import lax
from jax.experimental import pallas as pl
from jax.experimental.pallas import tpu as pltpu
from jax.experimental.pallas import tpu_sc as plsc

B = 8
A = 3
H = 64
W = 64
C = 80
N = 20
STRIDE = 8.0
ROW = 5 + C
P = B * A * H * W
BN = B * N
NWORDS = P * ROW
TAB = NWORDS // 16
GROW = 96

_INFO = plsc.get_sparse_core_info()
_NC = _INFO.num_cores
_NS = _INFO.num_subcores
_NW = _NC * _NS
_PPT = P // _NW
_CHUNK = 128
_NCHUNK = _PPT // _CHUNK
_JPT = BN // _NW


def _softplus(x):
    return jnp.maximum(x, 0.0) + jnp.log1p(jnp.exp(-jnp.abs(x)))


def _assign_body(boxes_ref, labels_ref, xs_ref, ys_ref, sz_ref,
                 pos_ref, win_ref, tx_ref, ty_ref, tw_ref, th_ref, npos_ref):
    bx = boxes_ref[...]
    x1 = bx[:, :, 0]
    y1 = bx[:, :, 1]
    x2 = bx[:, :, 2]
    y2 = bx[:, :, 3]
    cx = (x1 + x2) * 0.5
    cy = (y1 + y2) * 0.5
    gx = jnp.clip((cx / STRIDE).astype(jnp.int32), 0, W - 1)
    gy = jnp.clip((cy / STRIDE).astype(jnp.int32), 0, H - 1)

    lanx = lax.broadcasted_iota(jnp.int32, (B, N, W), 2)
    acx = jnp.sum(jnp.where(gx[:, :, None] == lanx, xs_ref[...][None], 0.0), axis=2)
    lany = lax.broadcasted_iota(jnp.int32, (B, N, H), 2)
    acy = jnp.sum(jnp.where(gy[:, :, None] == lany, ys_ref[...][None], 0.0), axis=2)

    area_g = (x2 - x1) * (y2 - y1)
    ious = []
    for a in range(A):
        aw = sz_ref[a, 0]
        ah = sz_ref[a, 1]
        ax1 = acx - aw * 0.5
        ay1 = acy - ah * 0.5
        ax2 = acx + aw * 0.5
        ay2 = acy + ah * 0.5
        iw = jnp.maximum(jnp.minimum(x2, ax2) - jnp.maximum(x1, ax1), 0.0)
        ih = jnp.maximum(jnp.minimum(y2, ay2) - jnp.maximum(y1, ay1), 0.0)
        inter = iw * ih
        area_a = (ax2 - ax1) * (ay2 - ay1)
        ious.append(inter / (area_g + area_a - inter + 1e-16))

    bi = jnp.maximum(jnp.maximum(ious[0], ious[1]), ious[2])
    ba = jnp.where(ious[0] == bi, 0,
                   jnp.where(ious[1] == bi, 1, 2)).astype(jnp.int32)

    key = ba * (H * W) + gy * W + gx
    ki = key[:, :, None]
    kj = key[:, None, :]
    bii = bi[:, :, None]
    bij = bi[:, None, :]
    ii = lax.broadcasted_iota(jnp.int32, (B, N, N), 1)
    jj = lax.broadcasted_iota(jnp.int32, (B, N, N), 2)
    beats = (ki == kj) & ((bij > bii) | ((bij == bii) & (jj < ii)))
    win = jnp.logical_not(jnp.any(beats, axis=2))

    aw_sel = jnp.where(ba == 0, sz_ref[0, 0],
                       jnp.where(ba == 1, sz_ref[1, 0], sz_ref[2, 0]))
    ah_sel = jnp.where(ba == 0, sz_ref[0, 1],
                       jnp.where(ba == 1, sz_ref[1, 1], sz_ref[2, 1]))
    gw = x2 - x1
    gh = y2 - y1

    bidx = lax.broadcasted_iota(jnp.int32, (B, N), 0)
    pos_ref[...] = bidx * (A * H * W) + key
    win_ref[...] = win.astype(jnp.float32)
    tx_ref[...] = (cx - acx) / STRIDE
    ty_ref[...] = (cy - acy) / STRIDE
    tw_ref[...] = jnp.log(gw / aw_sel + 1e-16)
    th_ref[...] = jnp.log(gh / ah_sel + 1e-16)
    npos_ref[0, 0] = jnp.sum(win.astype(jnp.float32))


def _sc_body(pred_ref, pos_ref, obj_ref, g_ref,
             idx_v, rows_v, objv, posv, pidx, prow, gloc, sem_o, sem_p):
    wid = lax.axis_index("s") * _NC + lax.axis_index("c")
    base = wid * _PPT
    i16 = lax.iota(jnp.int32, 16)

    def _build(c, _):
        for u in range(8):
            pos16 = base + c * _CHUNK + u * 16 + i16
            idx_v[pl.ds(c * _CHUNK + u * 16, 16)] = (pos16 * ROW + 4) >> 4
        return 0
    lax.fori_loop(0, _NCHUNK, _build, 0)

    ohandles = []
    for c in range(_NCHUNK):
        ohandles.append(pltpu.async_copy(
            pred_ref.at[idx_v.at[pl.ds(c * _CHUNK, _CHUNK)]],
            rows_v.at[pl.ds(c * _CHUNK, _CHUNK)], sem_o))

    pltpu.sync_copy(pos_ref, posv)
    phandles = []
    for m in range(_JPT):
        j = wid * _JPT + m
        pj = plsc.load_gather(posv, [jnp.zeros((16,), jnp.int32) + j])
        rb = (pj * ROW) >> 4
        pidx[pl.ds(m * 16, 16)] = jnp.minimum(rb + i16, TAB - 1)
    for m in range(_JPT):
        phandles.append(pltpu.async_copy(
            pred_ref.at[pidx.at[pl.ds(m * 16, 16)]],
            prow.at[pl.ds(m * 16, 16)], sem_p))

    for h in ohandles:
        h.wait()

    def _extract(c, _):
        p16 = base + c * 16 + i16
        s = p16 * ROW + 4
        vals = plsc.load_gather(rows_v, [c * 16 + i16, s & 15])
        objv[pl.ds(c * 16, 16)] = vals
        return 0
    lax.fori_loop(0, _PPT // 16, _extract, 0)
    pltpu.sync_copy(objv, obj_ref.at[pl.ds(base, _PPT)])

    for h in phandles:
        h.wait()
    for m in range(_JPT):
        j = wid * _JPT + m
        pj = plsc.load_gather(posv, [jnp.zeros((16,), jnp.int32) + j])
        off = (pj * ROW) & 15
        for c6 in range(GROW // 16):
            w = off + c6 * 16 + i16
            vals = plsc.load_gather(prow, [m * 16 + (w >> 4), w & 15])
            gloc[pl.ds(m * GROW + c6 * 16, 16)] = vals
    pltpu.sync_copy(gloc, g_ref.at[pl.ds(wid * _JPT * GROW, _JPT * GROW)])


def _final_body(obj_ref, g_ref, win_ref, tx_ref, ty_ref, tw_ref, th_ref,
                lab_ref, npos_ref, out_ref):
    s0 = jnp.sum(_softplus(obj_ref[...]))

    g = g_ref[...]
    lan = lax.broadcasted_iota(jnp.int32, (B, N, GROW), 2)
    g0 = jnp.sum(jnp.where(lan == 0, g, 0.0), axis=2)
    g1 = jnp.sum(jnp.where(lan == 1, g, 0.0), axis=2)
    g2 = jnp.sum(jnp.where(lan == 2, g, 0.0), axis=2)
    g3 = jnp.sum(jnp.where(lan == 3, g, 0.0), axis=2)
    g4 = jnp.sum(jnp.where(lan == 4, g, 0.0), axis=2)
    sp = jnp.sum(jnp.where((lan >= 5) & (lan < ROW), _softplus(g), 0.0), axis=2)
    glab = jnp.sum(jnp.where(lan == lab_ref[...][:, :, None] + 5, g, 0.0), axis=2)

    win = win_ref[...]
    box_sse = ((g0 - tx_ref[...]) ** 2 + (g1 - ty_ref[...]) ** 2
               + (g2 - tw_ref[...]) ** 2 + (g3 - th_ref[...]) ** 2)
    box_s = jnp.sum(win * box_sse)
    obj_corr = jnp.sum(win * g4)
    cls_s = jnp.sum(win * (sp - glab))

    npos = npos_ref[0, 0]
    obj_loss = (s0 - obj_corr) / float(P)
    npos_safe = jnp.where(npos > 0, npos, 1.0)
    box_loss = jnp.where(npos > 0, box_s / (npos_safe * 4.0), 0.0)
    cls_loss = jnp.where(npos > 0, cls_s / (npos_safe * float(C)), 0.0)
    out_ref[0, 0] = 0.05 * box_loss + obj_loss + 0.5 * cls_loss


def kernel(predictions, anchors, gt_boxes, gt_labels):
    f32 = jnp.float32
    xs = anchors[0, 0, :, 0][None, :]
    ys = anchors[0, :, 0, 1][None, :]
    sz = anchors[:, 0, 0, 2:4]
    labels = gt_labels.astype(jnp.int32)

    smem = pl.BlockSpec(memory_space=pltpu.SMEM)

    pos, win, tx, ty, tw, th, npos = pl.pallas_call(
        _assign_body,
        out_shape=[
            jax.ShapeDtypeStruct((B, N), jnp.int32),
            jax.ShapeDtypeStruct((B, N), f32),
            jax.ShapeDtypeStruct((B, N), f32),
            jax.ShapeDtypeStruct((B, N), f32),
            jax.ShapeDtypeStruct((B, N), f32),
            jax.ShapeDtypeStruct((B, N), f32),
            jax.ShapeDtypeStruct((1, 1), f32),
        ],
        in_specs=[pl.BlockSpec(), pl.BlockSpec(), pl.BlockSpec(),
                  pl.BlockSpec(), smem],
        out_specs=[pl.BlockSpec()] * 6 + [smem],
    )(gt_boxes, labels, xs, ys, sz)

    pred16 = predictions.reshape(TAB, 16)
    posf = pos.reshape(BN)

    sc_fn = functools.partial(
        pl.kernel,
        mesh=plsc.VectorSubcoreMesh(core_axis_name="c", subcore_axis_name="s"),
        compiler_params=pltpu.CompilerParams(needs_layout_passes=False,
                                             use_tc_tiling_on_sc=False),
        out_type=[
            jax.ShapeDtypeStruct((P,), f32),
            jax.ShapeDtypeStruct((BN * GROW,), f32),
        ],
        scratch_types=[
            pltpu.VMEM((_PPT,), jnp.int32),
            pltpu.VMEM((_PPT, 16), f32),
            pltpu.VMEM((_PPT,), f32),
            pltpu.VMEM((BN,), jnp.int32),
            pltpu.VMEM((_JPT * 16,), jnp.int32),
            pltpu.VMEM((_JPT * 16, 16), f32),
            pltpu.VMEM((_JPT * GROW,), f32),
            pltpu.SemaphoreType.DMA,
            pltpu.SemaphoreType.DMA,
        ],
    )(_sc_body)
    obj_col, g_flat = sc_fn(pred16, posf)

    g = g_flat.reshape(B, N, GROW)
    obj2 = obj_col.reshape(P // 128, 128)

    out = pl.pallas_call(
        _final_body,
        out_shape=jax.ShapeDtypeStruct((1, 1), f32),
        in_specs=[pl.BlockSpec()] * 8 + [smem],
        out_specs=smem,
    )(obj2, g, win, tx, ty, tw, th, labels, npos)
    return out[0, 0]


# speedup vs baseline: 1.8961x; 1.8961x over previous
import jax
import jax.numpy as jnp
from jax import lax
from jax.experimental import pallas as pl
from jax.experimental.pallas import tpu as pltpu

B = 8
A = 3
H = 64
W = 64
C = 80
N = 20
STRIDE = 8.0
ROW = 5 + C
P = B * A * H * W
BN = B * N
PR = P // 128
BH = 64
NSTEP = PR // BH

def _softplus(x):
    return jnp.maximum(x, 0.0) + jnp.log1p(jnp.exp(-jnp.abs(x)))


def _assign_body(boxes_ref, labels_ref, xs_ref, ys_ref, sz_ref,
                 pos_ref, win_ref, tx_ref, ty_ref, tw_ref, th_ref, npos_ref):
    bx = boxes_ref[...]
    x1 = bx[:, :, 0]
    y1 = bx[:, :, 1]
    x2 = bx[:, :, 2]
    y2 = bx[:, :, 3]
    cx = (x1 + x2) * 0.5
    cy = (y1 + y2) * 0.5
    gx = jnp.clip((cx / STRIDE).astype(jnp.int32), 0, W - 1)
    gy = jnp.clip((cy / STRIDE).astype(jnp.int32), 0, H - 1)

    lanx = lax.broadcasted_iota(jnp.int32, (B, N, W), 2)
    acx = jnp.sum(jnp.where(gx[:, :, None] == lanx, xs_ref[...][None], 0.0), axis=2)
    lany = lax.broadcasted_iota(jnp.int32, (B, N, H), 2)
    acy = jnp.sum(jnp.where(gy[:, :, None] == lany, ys_ref[...][None], 0.0), axis=2)

    area_g = (x2 - x1) * (y2 - y1)
    ious = []
    for a in range(A):
        aw = sz_ref[a, 0]
        ah = sz_ref[a, 1]
        ax1 = acx - aw * 0.5
        ay1 = acy - ah * 0.5
        ax2 = acx + aw * 0.5
        ay2 = acy + ah * 0.5
        iw = jnp.maximum(jnp.minimum(x2, ax2) - jnp.maximum(x1, ax1), 0.0)
        ih = jnp.maximum(jnp.minimum(y2, ay2) - jnp.maximum(y1, ay1), 0.0)
        inter = iw * ih
        area_a = (ax2 - ax1) * (ay2 - ay1)
        ious.append(inter / (area_g + area_a - inter + 1e-16))

    bi = jnp.maximum(jnp.maximum(ious[0], ious[1]), ious[2])
    ba = jnp.where(ious[0] == bi, 0,
                   jnp.where(ious[1] == bi, 1, 2)).astype(jnp.int32)

    key = ba * (H * W) + gy * W + gx
    ki = key[:, :, None]
    kj = key[:, None, :]
    bii = bi[:, :, None]
    bij = bi[:, None, :]
    ii = lax.broadcasted_iota(jnp.int32, (B, N, N), 1)
    jj = lax.broadcasted_iota(jnp.int32, (B, N, N), 2)
    beats = (ki == kj) & ((bij > bii) | ((bij == bii) & (jj < ii)))
    win = jnp.logical_not(jnp.any(beats, axis=2))

    aw_sel = jnp.where(ba == 0, sz_ref[0, 0],
                       jnp.where(ba == 1, sz_ref[1, 0], sz_ref[2, 0]))
    ah_sel = jnp.where(ba == 0, sz_ref[0, 1],
                       jnp.where(ba == 1, sz_ref[1, 1], sz_ref[2, 1]))
    gw = x2 - x1
    gh = y2 - y1

    bidx = lax.broadcasted_iota(jnp.int32, (B, N), 0)
    pos_ref[...] = bidx * (A * H * W) + key
    win_ref[...] = win.astype(jnp.float32)
    tx_ref[...] = (cx - acx) / STRIDE
    ty_ref[...] = (cy - acy) / STRIDE
    tw_ref[...] = jnp.log(gw / aw_sel + 1e-16)
    th_ref[...] = jnp.log(gh / ah_sel + 1e-16)
    npos_ref[0, 0] = jnp.sum(win.astype(jnp.float32))


def _sweep_body(pred_ref, s0_ref):
    i = pl.program_id(0)

    @pl.when(i == 0)
    def _():
        s0_ref[0, 0] = 0.0

    obj = pred_ref[:, :, 4]
    s0_ref[0, 0] += jnp.sum(_softplus(obj))


def _final_body(pred_hbm_ref, s0_ref, pos_ref, win_ref, tx_ref, ty_ref,
                tw_ref, th_ref, lab_ref, npos_ref, out_ref, rows, sem_r):
    def _row_dma(m):
        p = pos_ref[0, m]
        return pltpu.make_async_copy(pred_hbm_ref.at[p >> 7, p & 127, :],
                                     rows.at[m, :], sem_r)

    for m in range(BN):
        _row_dma(m).start()
    for m in range(BN):
        _row_dma(m).wait()

    s0 = s0_ref[0, 0]
    g = jnp.reshape(rows[...], (B, N, ROW))
    lan = lax.broadcasted_iota(jnp.int32, (B, N, ROW), 2)
    g0 = jnp.sum(jnp.where(lan == 0, g, 0.0), axis=2)
    g1 = jnp.sum(jnp.where(lan == 1, g, 0.0), axis=2)
    g2 = jnp.sum(jnp.where(lan == 2, g, 0.0), axis=2)
    g3 = jnp.sum(jnp.where(lan == 3, g, 0.0), axis=2)
    g4 = jnp.sum(jnp.where(lan == 4, g, 0.0), axis=2)
    sp = jnp.sum(jnp.where(lan >= 5, _softplus(g), 0.0), axis=2)
    glab = jnp.sum(jnp.where(lan == lab_ref[...][:, :, None] + 5, g, 0.0), axis=2)

    win = win_ref[...]
    box_sse = ((g0 - tx_ref[...]) ** 2 + (g1 - ty_ref[...]) ** 2
               + (g2 - tw_ref[...]) ** 2 + (g3 - th_ref[...]) ** 2)
    box_s = jnp.sum(win * box_sse)
    obj_corr = jnp.sum(win * g4)
    cls_s = jnp.sum(win * (sp - glab))

    npos = npos_ref[0, 0]
    obj_loss = (s0 - obj_corr) / float(P)
    npos_safe = jnp.where(npos > 0, npos, 1.0)
    box_loss = jnp.where(npos > 0, box_s / (npos_safe * 4.0), 0.0)
    cls_loss = jnp.where(npos > 0, cls_s / (npos_safe * float(C)), 0.0)
    out_ref[0, 0] = 0.05 * box_loss + obj_loss + 0.5 * cls_loss


def kernel(predictions, anchors, gt_boxes, gt_labels):
    f32 = jnp.float32
    xs = anchors[0, 0, :, 0][None, :]
    ys = anchors[0, :, 0, 1][None, :]
    sz = anchors[:, 0, 0, 2:4]
    labels = gt_labels.astype(jnp.int32)

    smem = pl.BlockSpec(memory_space=pltpu.SMEM)

    pos, win, tx, ty, tw, th, npos = pl.pallas_call(
        _assign_body,
        out_shape=[
            jax.ShapeDtypeStruct((B, N), jnp.int32),
            jax.ShapeDtypeStruct((B, N), f32),
            jax.ShapeDtypeStruct((B, N), f32),
            jax.ShapeDtypeStruct((B, N), f32),
            jax.ShapeDtypeStruct((B, N), f32),
            jax.ShapeDtypeStruct((B, N), f32),
            jax.ShapeDtypeStruct((1, 1), f32),
        ],
        in_specs=[pl.BlockSpec(), pl.BlockSpec(), pl.BlockSpec(),
                  pl.BlockSpec(), smem],
        out_specs=[pl.BlockSpec()] * 6 + [smem],
    )(gt_boxes, labels, xs, ys, sz)

    p3 = predictions.reshape(PR, 128, ROW)
    posf = pos.reshape(1, BN)

    s0 = pl.pallas_call(
        _sweep_body,
        grid=(NSTEP,),
        out_shape=jax.ShapeDtypeStruct((1, 1), f32),
        in_specs=[pl.BlockSpec((BH, 128, ROW), lambda i: (i, 0, 0))],
        out_specs=pl.BlockSpec(memory_space=pltpu.SMEM,
                               index_map=lambda i: (0, 0)),
    )(p3)

    out = pl.pallas_call(
        _final_body,
        out_shape=jax.ShapeDtypeStruct((1, 1), f32),
        in_specs=[pl.BlockSpec(memory_space=pltpu.MemorySpace.HBM),
                  smem, smem,
                  pl.BlockSpec(), pl.BlockSpec(), pl.BlockSpec(),
                  pl.BlockSpec(), pl.BlockSpec(), pl.BlockSpec(), smem],
        out_specs=smem,
        scratch_shapes=[
            pltpu.VMEM((BN, ROW), f32),
            pltpu.SemaphoreType.DMA,
        ],
    )(p3, s0, posf, win, tx, ty, tw, th, labels, npos)
    return out[0, 0]
